# baseline (speedup 1.0000x reference)
import functools

import jax
import jax.numpy as jnp
import numpy as np
from jax import lax
from jax.experimental import pallas as pl
from jax.experimental.pallas import tpu as pltpu, tpu_sc as plsc

N = 1_000_000
NX, NY = 432, 496
NUM_GRID = NX * NY
MAXV = 16_000
MAXP = 32
G0, G1, G2 = np.float32(0.0), np.float32(-39.68), np.float32(-3.0)
V0, V1, V2 = np.float32(0.16), np.float32(0.16), np.float32(4.0)

W = 32
CHUNK = 31_360
NP = W * CHUNK
SUB = 4_480
NSUB = 7
SUBROWS = SUB // 128
NROWS128 = NP // 128
RANGE = 6_704
NBINS_PAD = W * RANGE
PAD_BIN = NUM_GRID + 1
TILE_HSLICE = NBINS_PAD // 16
VOXROWS = MAXV * MAXP + 128
DUMP_ROW = MAXV * MAXP
COMBROWS = 16_064
DUMP_RANK = 16_001
BLK_ROWS = 98
NBLK = NROWS128 // BLK_ROWS
Q_CAP, Q_FLUSH = 4_224, 4_096
SQ_FLUSH = 1_024
SENT_U = RANGE + 15

_MESH = plsc.VectorSubcoreMesh(core_axis_name="c", subcore_axis_name="s")
_CPARAMS = pltpu.CompilerParams(needs_layout_passes=False,
                                use_tc_tiling_on_sc=False)
_CONST = {}


def _perm2d():
    if "perm" not in _CONST:
        cpu = jax.devices("cpu")[0]
        with jax.ensure_compile_time_eval(), jax.default_device(cpu):
            p = jax.random.permutation(jax.random.key(1), N)
        p = np.asarray(jax.device_get(p)).astype(np.int32)
        pad = np.arange(N, NP, dtype=np.int32)
        _CONST["perm"] = np.concatenate([p, pad]).reshape(NROWS128, 128)
    return _CONST["perm"]


def _wid():
    return lax.axis_index("c") * 16 + lax.axis_index("s")


def _iota():
    return lax.iota(jnp.int32, 16)


def _splat(v):
    return jnp.full((16,), v, jnp.int32)


def _floori(a):
    t = a.astype(jnp.int32)
    return t - (t.astype(jnp.float32) > a).astype(jnp.int32)


@functools.partial(
    pl.kernel,
    out_type=jax.ShapeDtypeStruct((NP, 16), jnp.float32),
    mesh=_MESH, compiler_params=_CPARAMS,
    scratch_types=[pltpu.VMEM((SUB, 4), jnp.float32),
                   pltpu.VMEM((SUB, 16), jnp.float32)],
)
def _k1a(points, packed, pts4, pk):
    w = _wid()
    iota = _iota()
    zcol = _splat(0)

    def compute(nv):
        def body(j, _):
            row = j * 16 + iota
            x = plsc.load_gather(pts4, [row, zcol])
            y = plsc.load_gather(pts4, [row, _splat(1)])
            z = plsc.load_gather(pts4, [row, _splat(2)])
            f = plsc.load_gather(pts4, [row, _splat(3)])
            cx = _floori((x - G0) / V0)
            cy = _floori((y - G1) / V1)
            cz = _floori((z - G2) / V2)
            ing = ((cx.astype(jnp.uint32) < jnp.uint32(NX))
                   & (cy.astype(jnp.uint32) < jnp.uint32(NY))
                   & (cz == 0))
            vid = jnp.where(ing, cy * NX + cx, NUM_GRID)
            plsc.store_scatter(pk, [row, zcol], x)
            plsc.store_scatter(pk, [row, _splat(1)], y)
            plsc.store_scatter(pk, [row, _splat(2)], z)
            plsc.store_scatter(pk, [row, _splat(3)], f)
            plsc.store_scatter(pk, [row, _splat(4)], plsc.bitcast(vid, jnp.float32))
            return 0
        lax.fori_loop(0, nv, body, 0)

    for k in range(NSUB):
        base = w * CHUNK + k * SUB
        full = base + SUB <= N

        @pl.when(full)
        def _(base=base):
            pltpu.sync_copy(points.at[pl.ds(base, SUB)], pts4)
            compute(SUB // 16)
            pltpu.sync_copy(pk, packed.at[pl.ds(base, SUB)])

        @pl.when(jnp.logical_not(full))
        def _(base=base):
            rem = 960
            pltpu.sync_copy(points.at[pl.ds(base, rem)], pts4.at[pl.ds(0, rem)])
            compute(rem // 16)
            pltpu.sync_copy(pk.at[pl.ds(0, rem)], packed.at[pl.ds(base, rem)])
            srow = jnp.where(iota == 4,
                             plsc.bitcast(_splat(PAD_BIN), jnp.float32),
                             jnp.zeros((16,), jnp.float32))

            def fill(r, _):
                pk[r, :] = srow
                return 0
            lax.fori_loop(0, NP - N, fill, 0)
            pltpu.sync_copy(pk.at[pl.ds(0, NP - N)], packed.at[pl.ds(N, NP - N)])


@functools.partial(
    pl.kernel,
    out_type=(jax.ShapeDtypeStruct((NP, 16), jnp.float32),
              jax.ShapeDtypeStruct((NROWS128, 128), jnp.int32),
              jax.ShapeDtypeStruct((2, NBINS_PAD), jnp.int32)),
    mesh=_MESH, compiler_params=_CPARAMS,
    scratch_types=[pltpu.VMEM((SUBROWS, 128), jnp.int32),
                   pltpu.VMEM((SUB, 16), jnp.float32),
                   pltpu.VMEM((SUBROWS, 128), jnp.int32),
                   pltpu.VMEM((128,), jnp.int32),
                   pltpu.VMEM((1024,), jnp.int32),
                   pltpu.VMEM_SHARED((NBINS_PAD,), jnp.int32),
                   pltpu.SemaphoreType.DMA],
)
def _k1b(packed, perm, ppts, vids, counts2, pidx, rows, vb, ones128, zb,
         hist, sem):
    c = lax.axis_index("c")
    s = lax.axis_index("s")
    w = c * 16 + s
    iota = _iota()

    for l in range(8):
        ones128[pl.ds(l * 16, 16)] = jnp.ones((16,), jnp.int32)
    for l in range(64):
        zb[pl.ds(l * 16, 16)] = jnp.zeros((16,), jnp.int32)
    hbase = s * TILE_HSLICE
    for t in range(13):
        pltpu.sync_copy(zb, hist.at[pl.ds(hbase + t * 1024, 1024)])
    pltpu.sync_copy(zb.at[pl.ds(0, 96)], hist.at[pl.ds(hbase + 13312, 96)])
    plsc.subcore_barrier()

    for k in range(NSUB):
        base = w * CHUNK + k * SUB
        grow = w * 245 + k * SUBROWS
        pltpu.sync_copy(perm.at[pl.ds(grow, SUBROWS)], pidx)
        descs = []
        for j in range(SUBROWS):
            descs.append(pltpu.async_copy(
                packed.at[pidx.at[j]], rows.at[pl.ds(j * 128, 128)], sem))
        for d in descs:
            d.wait()

        def body(t, _):
            j = t >> 3
            l = t & 7
            row = j * 128 + l * 16 + iota
            v = plsc.load_gather(rows, [row, _splat(4)])
            vb[j, pl.ds(l * 16, 16)] = plsc.bitcast(v, jnp.int32)
            return 0
        lax.fori_loop(0, SUB // 16, body, 0)

        pltpu.sync_copy(vb, vids.at[pl.ds(grow, SUBROWS)])
        pltpu.sync_copy(rows, ppts.at[pl.ds(base, SUB)])
        for j in range(SUBROWS):
            pltpu.sync_copy(ones128, hist.at[vb.at[j]], add=True)

    plsc.subcore_barrier()
    pltpu.sync_copy(hist.at[pl.ds(hbase, TILE_HSLICE)],
                    counts2.at[c, pl.ds(hbase, TILE_HSLICE)])


@functools.partial(
    pl.kernel,
    out_type=jax.ShapeDtypeStruct((W, 16), jnp.int32),
    mesh=_MESH, compiler_params=_CPARAMS,
    scratch_types=[pltpu.VMEM((RANGE,), jnp.int32),
                   pltpu.VMEM((RANGE,), jnp.int32),
                   pltpu.VMEM((16,), jnp.int32)],
)
def _k2a(counts2, totals, c0b, c1b, tv):
    w = _wid()
    iota = _iota()
    lo = w * RANGE
    pltpu.sync_copy(counts2.at[0, pl.ds(lo, RANGE)], c0b)
    pltpu.sync_copy(counts2.at[1, pl.ds(lo, RANGE)], c1b)

    def body(t, tot):
        cnt = c0b[pl.ds(t * 16, 16)] + c1b[pl.ds(t * 16, 16)]
        binv = lo + t * 16 + iota
        occ = (cnt > 0) & (binv < NUM_GRID)
        return tot + plsc.all_reduce_population_count(occ)

    tot = lax.fori_loop(0, RANGE // 16, body, jnp.zeros((16,), jnp.int32))
    tv[...] = tot
    pltpu.sync_copy(tv, totals.at[w])


def _rank_base(tot2, w):
    iota = _iota()
    t0 = plsc.load_gather(tot2, [iota, iota])
    t1 = plsc.load_gather(tot2, [iota + 16, iota])
    base = (jnp.sum(jnp.where(iota < w, t0, 0))
            + jnp.sum(jnp.where(iota + 16 < w, t1, 0)))
    mine = (jnp.sum(jnp.where(iota == w, t0, 0))
            + jnp.sum(jnp.where(iota + 16 == w, t1, 0)))
    gtot = jnp.sum(t0) + jnp.sum(t1)
    return base, mine, gtot


@functools.partial(
    pl.kernel,
    out_type=(jax.ShapeDtypeStruct((NBINS_PAD,), jnp.int32),
              jax.ShapeDtypeStruct((COMBROWS, 16), jnp.float32)),
    mesh=_MESH, compiler_params=_CPARAMS,
    scratch_types=[pltpu.VMEM((RANGE,), jnp.int32),
                   pltpu.VMEM((RANGE,), jnp.int32),
                   pltpu.VMEM((RANGE,), jnp.int32),
                   pltpu.VMEM((3456, 16), jnp.float32),
                   pltpu.VMEM((27, 128), jnp.int32),
                   pltpu.VMEM((W, 16), jnp.int32),
                   pltpu.VMEM((32, 16), jnp.float32)],
)
def _k2b(counts2, totals, rank_full, comb, c0b, c1b, rkb, crow, cidx, tot2,
         zb16):
    w = _wid()
    iota = _iota()
    lo = w * RANGE
    pltpu.sync_copy(totals, tot2)
    pltpu.sync_copy(counts2.at[0, pl.ds(lo, RANGE)], c0b)
    pltpu.sync_copy(counts2.at[1, pl.ds(lo, RANGE)], c1b)
    base, mine, gtot = _rank_base(tot2, w)

    zvec = jnp.zeros((16,), jnp.float32)

    def zrow(r, _):
        zb16[r, :] = zvec
        return 0
    lax.fori_loop(0, 32, zrow, 0)

    def czrow(r, _):
        crow[r, :] = zvec
        return 0
    lax.fori_loop(0, 3456, czrow, 0)

    def flush(cq):
        nb = (cq + 127) >> 7
        start = cq >> 4 << 4
        for t in range(8):
            bi = start + t * 16
            idx16 = bi + iota
            m = (idx16 >= cq) & (idx16 < nb * 128)
            plsc.store_scatter(cidx, [idx16 >> 7, idx16 & 127],
                               _splat(DUMP_RANK), mask=m)

        def fb(b, _):
            pltpu.sync_copy(crow.at[pl.ds(b * 128, 128)], comb.at[cidx.at[b]])
            return 0
        lax.fori_loop(0, nb, fb, 0)
        return jnp.int32(0)

    def body(t, carry):
        run, cq = carry
        cnt = c0b[pl.ds(t * 16, 16)] + c1b[pl.ds(t * 16, 16)]
        binv = lo + t * 16 + iota
        occ = (cnt > 0) & (binv < NUM_GRID)
        occi = occ.astype(jnp.int32)
        incl = plsc.cumsum(occi)
        rank = base + run + incl - occi
        keep = occ & (rank < MAXV)
        rkb[pl.ds(t * 16, 16)] = jnp.where(keep, rank, MAXV)
        ki = keep.astype(jnp.int32)
        kincl = plsc.cumsum(ki)
        cpos = cq + kincl - ki
        plsc.store_scatter(cidx, [cpos >> 7, cpos & 127], rank, mask=keep)
        yf = (binv // NX).astype(jnp.float32)
        xf = (binv % NX).astype(jnp.float32)
        cf = jnp.minimum(cnt, MAXP).astype(jnp.float32)
        plsc.store_scatter(crow, [cpos, _splat(1)], yf, mask=keep)
        plsc.store_scatter(crow, [cpos, _splat(2)], xf, mask=keep)
        plsc.store_scatter(crow, [cpos, _splat(4)], cf, mask=keep)
        run = run + incl[15]
        cq = cq + kincl[15]
        cq = lax.cond(cq >= 3328, flush, lambda q: q, cq)
        return run, cq

    _, cq = lax.fori_loop(0, RANGE // 16, body,
                          (jnp.int32(0), jnp.int32(0)))
    _ = lax.cond(cq > 0, flush, lambda q: jnp.int32(0), cq)
    pltpu.sync_copy(rkb, rank_full.at[pl.ds(lo, RANGE)])

    @pl.when(w == W - 1)
    def _():
        gk = jnp.minimum(gtot, MAXV)

        def z1(i, _):
            pltpu.sync_copy(zb16.at[pl.ds(0, 1)], comb.at[pl.ds(gk + i, 1)])
            return 0
        head = jnp.minimum((32 - (gk & 31)) & 31, COMBROWS - gk)
        lax.fori_loop(0, head, z1, 0)
        r0 = gk + head

        def z32(i, _):
            pltpu.sync_copy(zb16, comb.at[pl.ds(r0 + i * 32, 32)])
            return 0
        lax.fori_loop(0, (COMBROWS - r0) >> 5, z32, 0)


@functools.partial(
    pl.kernel,
    out_type=jax.ShapeDtypeStruct((VOXROWS, 16), jnp.float32),
    mesh=_MESH, compiler_params=_CPARAMS,
    scratch_types=[pltpu.VMEM((BLK_ROWS, 128), jnp.int32),
                   pltpu.VMEM((RANGE + 16,), jnp.int32),
                   pltpu.VMEM((RANGE + 16,), jnp.int32),
                   pltpu.VMEM((Q_CAP,), jnp.int32),
                   pltpu.VMEM((Q_CAP,), jnp.int32),
                   pltpu.VMEM((9, 128), jnp.int32),
                   pltpu.VMEM((9, 128), jnp.int32),
                   pltpu.VMEM((1152, 16), jnp.float32),
                   pltpu.VMEM((W, 16), jnp.int32),
                   pltpu.VMEM((1024, 16), jnp.float32),
                   pltpu.SemaphoreType.DMA],
)
def _k3(vids, rank_full, totals, ppts, vox, vblk, rank_l, cnts, qu, qp,
        sqd, sqp, spts, tot2, zb16, sem):
    w = _wid()
    iota = _iota()
    lo = w * RANGE
    pltpu.sync_copy(totals, tot2)
    base, mine, gtot = _rank_base(tot2, w)
    b_lo = jnp.minimum(base, MAXV)
    b_hi = jnp.minimum(base + mine, MAXV)

    zvec = jnp.zeros((16,), jnp.float32)

    def zrow(r, _):
        zb16[r, :] = zvec
        return 0
    lax.fori_loop(0, 1024, zrow, 0)

    r_start = b_lo * MAXP
    r_end = jnp.where(w == W - 1, jnp.int32(VOXROWS),
                      b_hi * MAXP)
    r_start = jnp.where(w == W - 1, jnp.minimum(r_start, gtot * 0 + r_start),
                        r_start)
    n1024 = (r_end - r_start) >> 10

    def zfull(i, _):
        pltpu.sync_copy(zb16, vox.at[pl.ds(r_start + i * 1024, 1024)])
        return 0
    lax.fori_loop(0, n1024, zfull, 0)
    r32 = r_start + n1024 * 1024

    def z32(i, _):
        pltpu.sync_copy(zb16.at[pl.ds(0, 32)], vox.at[pl.ds(r32 + i * 32, 32)])
        return 0
    lax.fori_loop(0, (r_end - r32) >> 5, z32, 0)

    pltpu.sync_copy(rank_full.at[pl.ds(lo, RANGE)], rank_l.at[pl.ds(0, RANGE)])
    rank_l[pl.ds(RANGE, 16)] = _splat(MAXV)

    def czero(t, _):
        cnts[pl.ds(t * 16, 16)] = jnp.zeros((16,), jnp.int32)
        return 0
    lax.fori_loop(0, (RANGE + 16) // 16, czero, 0)

    def sqflush(sqn):
        nb = (sqn + 127) >> 7
        start = sqn >> 4 << 4
        for t in range(8):
            idx16 = start + t * 16 + iota
            m = (idx16 >= sqn) & (idx16 < nb * 128)
            plsc.store_scatter(sqd, [idx16 >> 7, idx16 & 127],
                               _splat(DUMP_ROW), mask=m)
            plsc.store_scatter(sqp, [idx16 >> 7, idx16 & 127], _splat(0),
                               mask=m)

        def gb(b, _):
            pltpu.sync_copy(ppts.at[sqp.at[b]], spts.at[pl.ds(b * 128, 128)])
            return 0
        lax.fori_loop(0, nb, gb, 0)

        def sb(b, _):
            pltpu.sync_copy(spts.at[pl.ds(b * 128, 128)], vox.at[sqd.at[b]])
            return 0
        lax.fori_loop(0, nb, sb, 0)
        return jnp.int32(0)

    def qproc(k, sqn):
        qv = qu[pl.ds(k * 16, 16)]
        qpv = qp[pl.ds(k * 16, 16)]
        occ, lst = plsc.scan_count(qv)
        bs = plsc.load_gather(cnts, [qv])
        slot = bs + occ - 1
        plsc.store_scatter(cnts, [qv], bs + occ, mask=lst)
        rk = plsc.load_gather(rank_l, [qv])
        keep = (rk < MAXV) & (slot < MAXP)
        dst = rk * MAXP + slot
        ki = keep.astype(jnp.int32)
        kincl = plsc.cumsum(ki)
        cp = sqn + kincl - ki
        plsc.store_scatter(sqd, [cp >> 7, cp & 127], dst, mask=keep)
        plsc.store_scatter(sqp, [cp >> 7, cp & 127], qpv, mask=keep)
        sqn = sqn + kincl[15]
        return lax.cond(sqn >= SQ_FLUSH, sqflush, lambda x: x, sqn)

    def qflush(qn, sqn):
        nv = qn >> 4
        sqn = lax.fori_loop(0, nv, qproc, sqn)
        resid_u = qu[pl.ds(nv * 16, 16)]
        resid_p = qp[pl.ds(nv * 16, 16)]
        qu[pl.ds(0, 16)] = resid_u
        qp[pl.ds(0, 16)] = resid_p
        return qn & 15, sqn

    def scan_row(t, carry):
        qn, sqn = carry
        b = t // BLK_ROWS
        r = t % BLK_ROWS
        for l in range(8):
            v = vblk[r, pl.ds(l * 16, 16)]
            u = v - lo
            m = u.astype(jnp.uint32) < jnp.uint32(RANGE)

            @pl.when(jnp.any(m))
            def _(u=u, m=m, l=l):
                pos = t * 128 + l * 16 + iota
                plsc.store_compressed(qu.at[pl.ds(qn, 16)], u, mask=m)
                plsc.store_compressed(qp.at[pl.ds(qn, 16)], pos, mask=m)
            qn = qn + jnp.sum(m.astype(jnp.int32))
        return lax.cond(qn >= Q_FLUSH, lambda c: qflush(*c),
                        lambda c: c, (qn, sqn))

    def scan_blk(b, carry):
        pltpu.sync_copy(vids.at[pl.ds(b * BLK_ROWS, BLK_ROWS)], vblk)

        def row(r, cc):
            return scan_row(b * BLK_ROWS + r, cc)
        return lax.fori_loop(0, BLK_ROWS, row, carry)

    qn, sqn = lax.fori_loop(0, NBLK, scan_blk,
                            (jnp.int32(0), jnp.int32(0)))

    qu[pl.ds(qn, 16)] = _splat(SENT_U)
    qp[pl.ds(qn, 16)] = _splat(0)
    nv = (qn + 15) >> 4
    sqn = lax.fori_loop(0, nv, qproc, sqn)
    _ = lax.cond(sqn > 0, sqflush, lambda x: jnp.int32(0), sqn)


def _tc_vox_slice(vox):
    def body(i_ref, o_ref):
        o_ref[...] = i_ref[:, :4].reshape(125, MAXP, 4)

    return pl.pallas_call(
        body,
        grid=(128,),
        in_specs=[pl.BlockSpec((4000, 16), lambda g: (g, 0))],
        out_specs=pl.BlockSpec((125, MAXP, 4), lambda g: (g, 0, 0)),
        out_shape=jax.ShapeDtypeStruct((MAXV, MAXP, 4), jnp.float32),
    )(vox)


def _tc_comb_slice(comb):
    def body(i_ref, c_ref, n_ref):
        c_ref[...] = i_ref[:MAXV, :3]
        n_ref[...] = i_ref[:MAXV, 4:5]

    return pl.pallas_call(
        body,
        in_specs=[pl.BlockSpec((COMBROWS, 16), lambda: (0, 0))],
        out_specs=[pl.BlockSpec((MAXV, 3), lambda: (0, 0)),
                   pl.BlockSpec((MAXV, 1), lambda: (0, 0))],
        out_shape=[jax.ShapeDtypeStruct((MAXV, 3), jnp.float32),
                   jax.ShapeDtypeStruct((MAXV, 1), jnp.float32)],
    )(comb)


def kernel(point_clouds):
    perm = jnp.asarray(_perm2d())
    packed = _k1a(point_clouds)
    ppts, vids, counts2 = _k1b(packed, perm)
    totals = _k2a(counts2)
    rank_full, comb = _k2b(counts2, totals)
    vox = _k3(vids, rank_full, totals, ppts)
    voxels = _tc_vox_slice(vox)
    coordinates, num_points = _tc_comb_slice(comb)
    return voxels, coordinates, num_points.reshape(MAXV)


# speedup vs baseline: 1.7271x; 1.7271x over previous
import functools

import jax
import jax.numpy as jnp
import numpy as np
from jax import lax
from jax.experimental import pallas as pl
from jax.experimental.pallas import tpu as pltpu, tpu_sc as plsc

N = 1_000_000
NX, NY = 432, 496
NUM_GRID = NX * NY
MAXV = 16_000
MAXP = 32
G0, G1, G2 = np.float32(0.0), np.float32(-39.68), np.float32(-3.0)
V0, V1, V2 = np.float32(0.16), np.float32(0.16), np.float32(4.0)

W = 32
CHUNK = 31_360
NP = W * CHUNK
SUB = 4_480
NSUB = 7
SUBROWS = SUB // 128
NROWS128 = NP // 128
RANGE = 6_704
NBINS_PAD = W * RANGE
PAD_BIN = NUM_GRID + 1
TILE_HSLICE = NBINS_PAD // 16
VOXROWS = MAXV * MAXP + 128
DUMP_ROW = MAXV * MAXP
COMBROWS = 16_064
DUMP_RANK = 16_001
BLK_ROWS = 98
NBLK = NROWS128 // BLK_ROWS
Q_CAP, Q_FLUSH = 4_224, 4_096
SQ_FLUSH = 1_024
SENT_U = RANGE + 15

_MESH = plsc.VectorSubcoreMesh(core_axis_name="c", subcore_axis_name="s")
_CPARAMS = pltpu.CompilerParams(needs_layout_passes=False,
                                use_tc_tiling_on_sc=False)
_CONST = {}


def _perm2d():
    if "perm" not in _CONST:
        cpu = jax.devices("cpu")[0]
        with jax.ensure_compile_time_eval(), jax.default_device(cpu):
            p = jax.random.permutation(jax.random.key(1), N)
        p = np.asarray(jax.device_get(p)).astype(np.int32)
        pad = np.arange(N, NP, dtype=np.int32)
        _CONST["perm"] = np.concatenate([p, pad]).reshape(NROWS128, 128)
    return _CONST["perm"]


def _wid():
    return lax.axis_index("c") * 16 + lax.axis_index("s")


def _iota():
    return lax.iota(jnp.int32, 16)


def _splat(v):
    return jnp.full((16,), v, jnp.int32)


def _floori(a):
    t = a.astype(jnp.int32)
    return t - (t.astype(jnp.float32) > a).astype(jnp.int32)


@functools.partial(
    pl.kernel,
    out_type=jax.ShapeDtypeStruct((NP, 16), jnp.float32),
    mesh=_MESH, compiler_params=_CPARAMS,
    scratch_types=[pltpu.VMEM((4, SUB), jnp.float32),
                   pltpu.VMEM((SUB, 16), jnp.float32)],
)
def _k1a(points_t, packed, cb, pk):
    w = _wid()
    iota = _iota()
    zcol = _splat(0)

    def compute(nv):
        def body(j, _):
            row = j * 16 + iota
            x = cb[0, pl.ds(j * 16, 16)]
            y = cb[1, pl.ds(j * 16, 16)]
            z = cb[2, pl.ds(j * 16, 16)]
            f = cb[3, pl.ds(j * 16, 16)]
            cx = _floori((x - G0) / V0)
            cy = _floori((y - G1) / V1)
            cz = _floori((z - G2) / V2)
            ing = ((cx.astype(jnp.uint32) < jnp.uint32(NX))
                   & (cy.astype(jnp.uint32) < jnp.uint32(NY))
                   & (cz == 0))
            vid = jnp.where(ing, cy * NX + cx, NUM_GRID)
            plsc.store_scatter(pk, [row, zcol], x)
            plsc.store_scatter(pk, [row, _splat(1)], y)
            plsc.store_scatter(pk, [row, _splat(2)], z)
            plsc.store_scatter(pk, [row, _splat(3)], f)
            plsc.store_scatter(pk, [row, _splat(4)], plsc.bitcast(vid, jnp.float32))
            return 0
        lax.fori_loop(0, nv, body, 0)

    for k in range(NSUB):
        base = w * CHUNK + k * SUB
        full = base + SUB <= N

        @pl.when(full)
        def _(base=base):
            for col in range(4):
                pltpu.sync_copy(points_t.at[col, pl.ds(base, SUB)], cb.at[col])
            compute(SUB // 16)
            pltpu.sync_copy(pk, packed.at[pl.ds(base, SUB)])

        @pl.when(jnp.logical_not(full))
        def _(base=base):
            rem = 960
            for col in range(4):
                pltpu.sync_copy(points_t.at[col, pl.ds(base, rem)],
                                cb.at[col, pl.ds(0, rem)])
            compute(rem // 16)
            pltpu.sync_copy(pk.at[pl.ds(0, rem)], packed.at[pl.ds(base, rem)])
            srow = jnp.where(iota == 4,
                             plsc.bitcast(_splat(PAD_BIN), jnp.float32),
                             jnp.zeros((16,), jnp.float32))

            def fill(r, _):
                pk[r, :] = srow
                return 0
            lax.fori_loop(0, NP - N, fill, 0)
            pltpu.sync_copy(pk.at[pl.ds(0, NP - N)], packed.at[pl.ds(N, NP - N)])


@functools.partial(
    pl.kernel,
    out_type=(jax.ShapeDtypeStruct((NP, 16), jnp.float32),
              jax.ShapeDtypeStruct((NROWS128, 128), jnp.int32),
              jax.ShapeDtypeStruct((2, NBINS_PAD), jnp.int32)),
    mesh=_MESH, compiler_params=_CPARAMS,
    scratch_types=[pltpu.VMEM((SUBROWS, 128), jnp.int32),
                   pltpu.VMEM((SUB, 16), jnp.float32),
                   pltpu.VMEM((SUBROWS, 128), jnp.int32),
                   pltpu.VMEM((128,), jnp.int32),
                   pltpu.VMEM((1024,), jnp.int32),
                   pltpu.VMEM_SHARED((NBINS_PAD,), jnp.int32),
                   pltpu.SemaphoreType.DMA],
)
def _k1b(packed, perm, ppts, vids, counts2, pidx, rows, vb, ones128, zb,
         hist, sem):
    c = lax.axis_index("c")
    s = lax.axis_index("s")
    w = c * 16 + s
    iota = _iota()

    for l in range(8):
        ones128[pl.ds(l * 16, 16)] = jnp.ones((16,), jnp.int32)
    for l in range(64):
        zb[pl.ds(l * 16, 16)] = jnp.zeros((16,), jnp.int32)
    hbase = s * TILE_HSLICE
    for t in range(13):
        pltpu.sync_copy(zb, hist.at[pl.ds(hbase + t * 1024, 1024)])
    pltpu.sync_copy(zb.at[pl.ds(0, 96)], hist.at[pl.ds(hbase + 13312, 96)])
    plsc.subcore_barrier()

    for k in range(NSUB):
        base = w * CHUNK + k * SUB
        grow = w * 245 + k * SUBROWS
        pltpu.sync_copy(perm.at[pl.ds(grow, SUBROWS)], pidx)
        descs = []
        for j in range(SUBROWS):
            descs.append(pltpu.async_copy(
                packed.at[pidx.at[j]], rows.at[pl.ds(j * 128, 128)], sem))
        for d in descs:
            d.wait()

        def body(t, _):
            j = t >> 3
            l = t & 7
            row = j * 128 + l * 16 + iota
            v = plsc.load_gather(rows, [row, _splat(4)])
            vb[j, pl.ds(l * 16, 16)] = plsc.bitcast(v, jnp.int32)
            return 0
        lax.fori_loop(0, SUB // 16, body, 0)

        pltpu.sync_copy(vb, vids.at[pl.ds(grow, SUBROWS)])
        pltpu.sync_copy(rows, ppts.at[pl.ds(base, SUB)])
        for j in range(SUBROWS):
            pltpu.sync_copy(ones128, hist.at[vb.at[j]], add=True)

    plsc.subcore_barrier()
    pltpu.sync_copy(hist.at[pl.ds(hbase, TILE_HSLICE)],
                    counts2.at[c, pl.ds(hbase, TILE_HSLICE)])


@functools.partial(
    pl.kernel,
    out_type=jax.ShapeDtypeStruct((W, 16), jnp.int32),
    mesh=_MESH, compiler_params=_CPARAMS,
    scratch_types=[pltpu.VMEM((RANGE,), jnp.int32),
                   pltpu.VMEM((RANGE,), jnp.int32),
                   pltpu.VMEM((16,), jnp.int32)],
)
def _k2a(counts2, totals, c0b, c1b, tv):
    w = _wid()
    iota = _iota()
    lo = w * RANGE
    pltpu.sync_copy(counts2.at[0, pl.ds(lo, RANGE)], c0b)
    pltpu.sync_copy(counts2.at[1, pl.ds(lo, RANGE)], c1b)

    def body(t, tot):
        cnt = c0b[pl.ds(t * 16, 16)] + c1b[pl.ds(t * 16, 16)]
        binv = lo + t * 16 + iota
        occ = (cnt > 0) & (binv < NUM_GRID)
        return tot + plsc.all_reduce_population_count(occ)

    tot = lax.fori_loop(0, RANGE // 16, body, jnp.zeros((16,), jnp.int32))
    tv[...] = tot
    pltpu.sync_copy(tv, totals.at[w])


def _rank_base(tot2, w):
    iota = _iota()
    t0 = plsc.load_gather(tot2, [iota, iota])
    t1 = plsc.load_gather(tot2, [iota + 16, iota])
    base = (jnp.sum(jnp.where(iota < w, t0, 0))
            + jnp.sum(jnp.where(iota + 16 < w, t1, 0)))
    mine = (jnp.sum(jnp.where(iota == w, t0, 0))
            + jnp.sum(jnp.where(iota + 16 == w, t1, 0)))
    gtot = jnp.sum(t0) + jnp.sum(t1)
    return base, mine, gtot


@functools.partial(
    pl.kernel,
    out_type=(jax.ShapeDtypeStruct((NBINS_PAD,), jnp.int32),
              jax.ShapeDtypeStruct((COMBROWS, 16), jnp.float32)),
    mesh=_MESH, compiler_params=_CPARAMS,
    scratch_types=[pltpu.VMEM((RANGE,), jnp.int32),
                   pltpu.VMEM((RANGE,), jnp.int32),
                   pltpu.VMEM((RANGE,), jnp.int32),
                   pltpu.VMEM((3456, 16), jnp.float32),
                   pltpu.VMEM((27, 128), jnp.int32),
                   pltpu.VMEM((W, 16), jnp.int32),
                   pltpu.VMEM((32, 16), jnp.float32)],
)
def _k2b(counts2, totals, rank_full, comb, c0b, c1b, rkb, crow, cidx, tot2,
         zb16):
    w = _wid()
    iota = _iota()
    lo = w * RANGE
    pltpu.sync_copy(totals, tot2)
    pltpu.sync_copy(counts2.at[0, pl.ds(lo, RANGE)], c0b)
    pltpu.sync_copy(counts2.at[1, pl.ds(lo, RANGE)], c1b)
    base, mine, gtot = _rank_base(tot2, w)

    zvec = jnp.zeros((16,), jnp.float32)

    def zrow(r, _):
        zb16[r, :] = zvec
        return 0
    lax.fori_loop(0, 32, zrow, 0)

    def czrow(r, _):
        crow[r, :] = zvec
        return 0
    lax.fori_loop(0, 3456, czrow, 0)

    def flush(cq):
        nb = (cq + 127) >> 7
        start = cq >> 4 << 4
        for t in range(8):
            bi = start + t * 16
            idx16 = bi + iota
            m = (idx16 >= cq) & (idx16 < nb * 128)
            plsc.store_scatter(cidx, [idx16 >> 7, idx16 & 127],
                               _splat(DUMP_RANK), mask=m)

        def fb(b, _):
            pltpu.sync_copy(crow.at[pl.ds(b * 128, 128)], comb.at[cidx.at[b]])
            return 0
        lax.fori_loop(0, nb, fb, 0)
        return jnp.int32(0)

    def body(t, carry):
        run, cq = carry
        cnt = c0b[pl.ds(t * 16, 16)] + c1b[pl.ds(t * 16, 16)]
        binv = lo + t * 16 + iota
        occ = (cnt > 0) & (binv < NUM_GRID)
        occi = occ.astype(jnp.int32)
        incl = plsc.cumsum(occi)
        rank = base + run + incl - occi
        keep = occ & (rank < MAXV)
        rkb[pl.ds(t * 16, 16)] = jnp.where(keep, rank, MAXV)
        ki = keep.astype(jnp.int32)
        kincl = plsc.cumsum(ki)
        cpos = cq + kincl - ki
        plsc.store_scatter(cidx, [cpos >> 7, cpos & 127], rank, mask=keep)
        yf = (binv // NX).astype(jnp.float32)
        xf = (binv % NX).astype(jnp.float32)
        cf = jnp.minimum(cnt, MAXP).astype(jnp.float32)
        plsc.store_scatter(crow, [cpos, _splat(1)], yf, mask=keep)
        plsc.store_scatter(crow, [cpos, _splat(2)], xf, mask=keep)
        plsc.store_scatter(crow, [cpos, _splat(4)], cf, mask=keep)
        run = run + incl[15]
        cq = cq + kincl[15]
        cq = lax.cond(cq >= 3328, flush, lambda q: q, cq)
        return run, cq

    _, cq = lax.fori_loop(0, RANGE // 16, body,
                          (jnp.int32(0), jnp.int32(0)))
    _ = lax.cond(cq > 0, flush, lambda q: jnp.int32(0), cq)
    pltpu.sync_copy(rkb, rank_full.at[pl.ds(lo, RANGE)])

    @pl.when(w == W - 1)
    def _():
        gk = jnp.minimum(gtot, MAXV)

        def z1(i, _):
            pltpu.sync_copy(zb16.at[pl.ds(0, 1)], comb.at[pl.ds(gk + i, 1)])
            return 0
        head = jnp.minimum((32 - (gk & 31)) & 31, COMBROWS - gk)
        lax.fori_loop(0, head, z1, 0)
        r0 = gk + head

        def z32(i, _):
            pltpu.sync_copy(zb16, comb.at[pl.ds(r0 + i * 32, 32)])
            return 0
        lax.fori_loop(0, (COMBROWS - r0) >> 5, z32, 0)


@functools.partial(
    pl.kernel,
    out_type=jax.ShapeDtypeStruct((VOXROWS, 16), jnp.float32),
    mesh=_MESH, compiler_params=_CPARAMS,
    scratch_types=[pltpu.VMEM((BLK_ROWS, 128), jnp.int32),
                   pltpu.VMEM((RANGE + 16,), jnp.int32),
                   pltpu.VMEM((RANGE + 16,), jnp.int32),
                   pltpu.VMEM((Q_CAP,), jnp.int32),
                   pltpu.VMEM((Q_CAP,), jnp.int32),
                   pltpu.VMEM((9, 128), jnp.int32),
                   pltpu.VMEM((9, 128), jnp.int32),
                   pltpu.VMEM((1152, 16), jnp.float32),
                   pltpu.VMEM((W, 16), jnp.int32),
                   pltpu.VMEM((1024, 16), jnp.float32),
                   pltpu.SemaphoreType.DMA],
)
def _k3(vids, rank_full, totals, ppts, vox, vblk, rank_l, cnts, qu, qp,
        sqd, sqp, spts, tot2, zb16, sem):
    c = lax.axis_index("c")
    s = lax.axis_index("s")
    w = c * 16 + s
    iota = _iota()
    lo = w * RANGE
    pltpu.sync_copy(totals, tot2)

    zvec = jnp.zeros((16,), jnp.float32)

    def zrow(r, _):
        zb16[r, :] = zvec
        return 0
    lax.fori_loop(0, 1024, zrow, 0)

    t0 = plsc.load_gather(tot2, [iota, iota])
    x16 = jnp.minimum(jnp.sum(t0), MAXV)
    row_split = x16 * MAXP
    sc_lo = jnp.where(c == 0, 0, row_split)
    sc_hi = jnp.where(c == 0, row_split, jnp.int32(VOXROWS))
    blocks32 = (sc_hi - sc_lo) >> 5
    per = (blocks32 + 15) >> 4
    my_lo = sc_lo + jnp.minimum(s * per, blocks32) * 32
    my_hi = sc_lo + jnp.minimum((s + 1) * per, blocks32) * 32
    n1024 = (my_hi - my_lo) >> 10

    def zfull(i, _):
        pltpu.sync_copy(zb16, vox.at[pl.ds(my_lo + i * 1024, 1024)])
        return 0
    lax.fori_loop(0, n1024, zfull, 0)
    r32 = my_lo + n1024 * 1024

    def z32(i, _):
        pltpu.sync_copy(zb16.at[pl.ds(0, 32)], vox.at[pl.ds(r32 + i * 32, 32)])
        return 0
    lax.fori_loop(0, (my_hi - r32) >> 5, z32, 0)
    plsc.subcore_barrier()

    pltpu.sync_copy(rank_full.at[pl.ds(lo, RANGE)], rank_l.at[pl.ds(0, RANGE)])
    rank_l[pl.ds(RANGE, 16)] = _splat(MAXV)

    def czero(t, _):
        cnts[pl.ds(t * 16, 16)] = jnp.zeros((16,), jnp.int32)
        return 0
    lax.fori_loop(0, (RANGE + 16) // 16, czero, 0)

    for b9 in range(9):
        for l9 in range(8):
            sqd[b9, pl.ds(l9 * 16, 16)] = _splat(DUMP_ROW)
            sqp[b9, pl.ds(l9 * 16, 16)] = _splat(0)

    def sqflush(sqn):
        gds = [pltpu.async_copy(ppts.at[sqp.at[b]],
                                spts.at[pl.ds(b * 128, 128)], sem)
               for b in range(9)]
        for d in gds:
            d.wait()
        sds = [pltpu.async_copy(spts.at[pl.ds(b * 128, 128)],
                                vox.at[sqd.at[b]], sem)
               for b in range(9)]
        for d in sds:
            d.wait()
        return jnp.int32(0)

    def qproc(k, sqn):
        qv = qu[pl.ds(k * 16, 16)]
        qpv = qp[pl.ds(k * 16, 16)]
        occ, lst = plsc.scan_count(qv)
        bs = plsc.load_gather(cnts, [qv])
        slot = bs + occ - 1
        plsc.store_scatter(cnts, [qv], bs + occ, mask=lst)
        rk = plsc.load_gather(rank_l, [qv])
        keep = (rk < MAXV) & (slot < MAXP)
        dst = rk * MAXP + slot
        ki = keep.astype(jnp.int32)
        kincl = plsc.cumsum(ki)
        cp = sqn + kincl - ki
        plsc.store_scatter(sqd, [cp >> 7, cp & 127], dst, mask=keep)
        plsc.store_scatter(sqp, [cp >> 7, cp & 127], qpv, mask=keep)
        sqn = sqn + kincl[15]
        return lax.cond(sqn >= SQ_FLUSH, sqflush, lambda x: x, sqn)

    def qflush(qn, sqn):
        nv = qn >> 4
        sqn = lax.fori_loop(0, nv, qproc, sqn)
        resid_u = qu[pl.ds(nv * 16, 16)]
        resid_p = qp[pl.ds(nv * 16, 16)]
        qu[pl.ds(0, 16)] = resid_u
        qp[pl.ds(0, 16)] = resid_p
        return qn & 15, sqn

    def scan_row(b, r, carry):
        qn, sqn = carry
        t = b * BLK_ROWS + r
        for l in range(8):
            v = vblk[r, pl.ds(l * 16, 16)]
            u = v - lo
            m = u.astype(jnp.uint32) < jnp.uint32(RANGE)

            @pl.when(jnp.any(m))
            def _(u=u, m=m, l=l):
                pos = t * 128 + l * 16 + iota
                plsc.store_compressed(qu.at[pl.ds(qn, 16)], u, mask=m)
                plsc.store_compressed(qp.at[pl.ds(qn, 16)], pos, mask=m)
            qn = qn + jnp.sum(m.astype(jnp.int32))
        return lax.cond(qn >= Q_FLUSH, lambda c: qflush(*c),
                        lambda c: c, (qn, sqn))

    def scan_blk(b, carry):
        pltpu.sync_copy(vids.at[pl.ds(b * BLK_ROWS, BLK_ROWS)], vblk)

        def row(r, cc):
            return scan_row(b, r, cc)
        return lax.fori_loop(0, BLK_ROWS, row, carry)

    qn, sqn = lax.fori_loop(0, NBLK, scan_blk,
                            (jnp.int32(0), jnp.int32(0)))

    qu[pl.ds(qn, 16)] = _splat(SENT_U)
    qp[pl.ds(qn, 16)] = _splat(0)
    nv = (qn + 15) >> 4
    sqn = lax.fori_loop(0, nv, qproc, sqn)
    _ = lax.cond(sqn > 0, sqflush, lambda x: jnp.int32(0), sqn)


def _tc_vox_slice(vox):
    def body(i_ref, o_ref):
        o_ref[...] = i_ref[:, :4].reshape(125, MAXP, 4)

    return pl.pallas_call(
        body,
        grid=(128,),
        in_specs=[pl.BlockSpec((4000, 16), lambda g: (g, 0))],
        out_specs=pl.BlockSpec((125, MAXP, 4), lambda g: (g, 0, 0)),
        out_shape=jax.ShapeDtypeStruct((MAXV, MAXP, 4), jnp.float32),
    )(vox)


def _tc_comb_slice(comb):
    def body(i_ref, c_ref, n_ref):
        c_ref[...] = i_ref[:MAXV, :3]
        n_ref[...] = i_ref[:MAXV, 4:5]

    return pl.pallas_call(
        body,
        in_specs=[pl.BlockSpec((COMBROWS, 16), lambda: (0, 0))],
        out_specs=[pl.BlockSpec((MAXV, 3), lambda: (0, 0)),
                   pl.BlockSpec((MAXV, 1), lambda: (0, 0))],
        out_shape=[jax.ShapeDtypeStruct((MAXV, 3), jnp.float32),
                   jax.ShapeDtypeStruct((MAXV, 1), jnp.float32)],
    )(comb)


def kernel(point_clouds):
    perm = jnp.asarray(_perm2d())
    packed = _k1a(point_clouds.T)
    ppts, vids, counts2 = _k1b(packed, perm)
    totals = _k2a(counts2)
    rank_full, comb = _k2b(counts2, totals)
    vox = _k3(vids, rank_full, totals, ppts)
    voxels = _tc_vox_slice(vox)
    coordinates, num_points = _tc_comb_slice(comb)
    return voxels, coordinates, num_points.reshape(MAXV)
